# baseline (speedup 1.0000x reference)
import jax
import jax.numpy as jnp
from jax import lax
from jax.experimental import pallas as pl
from jax.experimental.pallas import tpu as pltpu
from jax.experimental.pallas import tpu_sc as plsc

N_BATCH = 100000
DIM = 64
NROWS = 16384
R = 12800
NPASS = 4
SCR = NPASS * R
SROWS = R + 264
CWORDS = SCR + 1280
CSLICE = CWORDS // 16
RPT = NROWS // 16
NPIECE = RPT // 16
NBATCH = RPT // 128
CHUNK = 80
NCHUNK = R // CHUNK
JMAX = NCHUNK // 16


def _body(zb_hbm, ids_hbm, cent_hbm, out_hbm,
          S_sh, C1_sh,
          ids_v, cidx2_v, ctgt2_v, cbuf_v, zeros_v, ones1_v,
          zln_v, sdr_v, cdr1_v, cc_v,
          gs0, gs1, ss0, ss1, sn0, sn1, sc0, sc1, sw0, sw1):
    c = lax.axis_index("c")
    s = lax.axis_index("s")
    row0 = s * RPT
    lane = lax.iota(jnp.int32, 16)
    zero16 = jnp.zeros((16,), jnp.float32)
    zero16i = jnp.zeros((16,), jnp.int32)
    one16 = jnp.full((16,), 1.0, jnp.float32)
    gsem = (gs0, gs1)
    ssem = (ss0, ss1)
    nsem = (sn0, sn1)
    csem = (sc0, sc1)
    wsem = (sw0, sw1)

    pltpu.sync_copy(ids_hbm.at[pl.ds(row0, RPT)], ids_v)

    def _init(i, _):
        for g in range(4):
            zeros_v[i, pl.ds(16 * g, 16)] = zero16

        @pl.when(i < 8)
        def _():
            ones1_v[pl.ds(i * 16, 16)] = one16

        @pl.when(i < 103)
        def _():
            zln_v[pl.ds(i * 16, 16)] = zero16
        return 0

    lax.fori_loop(0, 128, _init, 0)

    pltpu.sync_copy(zln_v.at[pl.ds(0, 1640)],
                    C1_sh.at[pl.ds(s * CSLICE, 1640)])
    pltpu.sync_copy(zln_v.at[pl.ds(0, 1640)],
                    C1_sh.at[pl.ds(s * CSLICE + 1640, 1640)])

    cbase = c * SCR

    def _pfc(j, _):
        for tt in range(8):
            kk = j * 8 + tt
            ctgt2_v[j, pl.ds(16 * tt, 16)] = SCR + ((kk % 64) * 16) + lane
        return 0

    lax.fori_loop(0, NBATCH + 1, _pfc, 0)
    cur = zero16i
    for k in range(NPIECE):
        ids16 = ids_v[pl.ds(16 * k, 16)]
        m = (ids16 >= cbase) & (ids16 < cbase + SCR)
        mi = jnp.where(m, 1, 0).astype(jnp.int32)
        pos = cur + lax.cumsum(mi) - mi
        plsc.store_scatter(ctgt2_v, [pos >> 7, pos & 127],
                           ids16 - cbase, mask=m)
        cur = cur + plsc.all_reduce_population_count(m)
    nC = cur[0]

    plsc.subcore_barrier()

    for b in range(NBATCH):
        @pl.when(b * 128 < nC)
        def _():
            pltpu.sync_copy(ones1_v, C1_sh.at[ctgt2_v.at[b]], add=True)

    wbp = [None, None]

    for r in range(NPASS):
        base = cbase + r * R

        def _pfp(j, _):
            for tt in range(8):
                kk = j * 8 + tt
                cidx2_v[j, pl.ds(16 * tt, 16)] = zero16i
                ctgt2_v[j, pl.ds(16 * tt, 16)] = R + ((kk % 14) * 16) + lane
            return 0

        lax.fori_loop(0, NBATCH + 1, _pfp, 0)

        cursor = zero16i
        for k in range(NPIECE):
            ids16 = ids_v[pl.ds(16 * k, 16)]
            m = (ids16 >= base) & (ids16 < base + R)
            mi = jnp.where(m, 1, 0).astype(jnp.int32)
            pos = cursor + lax.cumsum(mi) - mi
            plsc.store_scatter(cidx2_v, [pos >> 7, pos & 127],
                               row0 + 16 * k + lane, mask=m)
            plsc.store_scatter(ctgt2_v, [pos >> 7, pos & 127],
                               ids16 - base, mask=m)
            cursor = cursor + plsc.all_reduce_population_count(m)
        n = cursor[0]

        for b in range(NBATCH):
            @pl.when(b * 128 < n)
            def _():
                pltpu.sync_copy(zeros_v, S_sh.at[ctgt2_v.at[b]])
        plsc.subcore_barrier()

        for b in range(NBATCH):
            @pl.when(b * 128 < n)
            def _():
                pltpu.sync_copy(zb_hbm.at[cidx2_v.at[b]], cbuf_v)
                pltpu.sync_copy(cbuf_v, S_sh.at[ctgt2_v.at[b]], add=True)
        plsc.subcore_barrier()

        def _drain(j, _):
            chunk = s + 16 * j

            @pl.when((chunk < NCHUNK) & (base + chunk * CHUNK < N_BATCH))
            def _():
                st = chunk * CHUNK
                pltpu.sync_copy(S_sh.at[pl.ds(st, CHUNK)], sdr_v)
                pltpu.sync_copy(C1_sh.at[pl.ds(r * R + st, CHUNK)], cdr1_v)
                pltpu.sync_copy(cent_hbm.at[pl.ds(base + st, CHUNK)],
                                cc_v.at[0])

                def _grp(ii, _):
                    rb = ii * 16
                    cnt16 = cdr1_v[pl.ds(rb, 16)]
                    inv16 = 0.1 / jnp.maximum(cnt16, 1.0)
                    pf16 = jnp.where(cnt16 > 0.0, 1.0, 0.0)
                    for l in range(16):
                        p = pf16[l] > 0.5
                        iv = inv16[l]
                        for g in range(4):
                            sv = sdr_v[rb + l, pl.ds(16 * g, 16)]
                            cv = cc_v[0, rb + l, pl.ds(16 * g, 16)]
                            cc_v[0, rb + l, pl.ds(16 * g, 16)] = jnp.where(
                                p, 0.9 * cv + iv * sv, cv)
                    return 0

                lax.fori_loop(0, CHUNK // 16, _grp, 0)
                pltpu.sync_copy(cc_v.at[0], out_hbm.at[pl.ds(base + st, CHUNK)])
            return 0

        lax.fori_loop(0, (NCHUNK + 15) // 16, _drain, 0)

        if r != NPASS - 1:
            plsc.subcore_barrier()

    for q in range(2):
        if wbp[q] is not None:
            h, cond = wbp[q]

            @pl.when(cond)
            def _():
                h.wait()


def kernel(zb, batch_ids, centers):
    mesh = plsc.VectorSubcoreMesh(core_axis_name="c", subcore_axis_name="s")
    run = pl.kernel(
        _body,
        out_type=jax.ShapeDtypeStruct((N_BATCH, DIM), jnp.float32),
        mesh=mesh,
        compiler_params=pltpu.CompilerParams(
            use_tc_tiling_on_sc=False, needs_layout_passes=False),
        scratch_types=[
            pltpu.VMEM_SHARED((SROWS, DIM), jnp.float32),
            pltpu.VMEM_SHARED((CWORDS,), jnp.float32),
            pltpu.VMEM((RPT,), jnp.int32),
            pltpu.VMEM((NBATCH + 1, 128), jnp.int32),
            pltpu.VMEM((NBATCH + 1, 128), jnp.int32),
            pltpu.VMEM((128, DIM), jnp.float32),
            pltpu.VMEM((128, DIM), jnp.float32),
            pltpu.VMEM((128,), jnp.float32),
            pltpu.VMEM((1648,), jnp.float32),
            pltpu.VMEM((CHUNK, DIM), jnp.float32),
            pltpu.VMEM((CHUNK,), jnp.float32),
            pltpu.VMEM((2, CHUNK, DIM), jnp.float32),
            pltpu.SemaphoreType.DMA,
            pltpu.SemaphoreType.DMA,
            pltpu.SemaphoreType.DMA,
            pltpu.SemaphoreType.DMA,
            pltpu.SemaphoreType.DMA,
            pltpu.SemaphoreType.DMA,
            pltpu.SemaphoreType.DMA,
            pltpu.SemaphoreType.DMA,
            pltpu.SemaphoreType.DMA,
            pltpu.SemaphoreType.DMA,
        ],
    )
    return run(zb, batch_ids.astype(jnp.int32), centers)


# speedup vs baseline: 1.0405x; 1.0405x over previous
import jax
import jax.numpy as jnp
from jax import lax
from jax.experimental import pallas as pl
from jax.experimental.pallas import tpu as pltpu
from jax.experimental.pallas import tpu_sc as plsc

N_BATCH = 100000
DIM = 64
NROWS = 16384
R = 12800
NPASS = 4
SCR = NPASS * R
SROWS = R + 264
CWORDS = SCR + 1280
CSLICE = CWORDS // 16
RPT = NROWS // 16
NPIECE = RPT // 16
NBATCH = RPT // 128
CHUNK = 160
NCHUNK = R // CHUNK
JMAX = NCHUNK // 16


def _body(zb_hbm, ids_hbm, cent_hbm, out_hbm,
          S_sh, C1_sh,
          ids_v, cidx2_v, ctgt2_v, cbuf_v, zeros_v, ones1_v,
          zln_v, sdr_v, cdr1_v, cc_v,
          gs0, gs1, ss0, ss1, sn0, sn1, sc0, sc1, sw0, sw1):
    c = lax.axis_index("c")
    s = lax.axis_index("s")
    row0 = s * RPT
    lane = lax.iota(jnp.int32, 16)
    zero16 = jnp.zeros((16,), jnp.float32)
    zero16i = jnp.zeros((16,), jnp.int32)
    one16 = jnp.full((16,), 1.0, jnp.float32)
    gsem = (gs0, gs1)
    ssem = (ss0, ss1)
    nsem = (sn0, sn1)
    csem = (sc0, sc1)
    wsem = (sw0, sw1)

    pltpu.sync_copy(ids_hbm.at[pl.ds(row0, RPT)], ids_v)

    def _init(i, _):
        for g in range(4):
            zeros_v[i, pl.ds(16 * g, 16)] = zero16

        @pl.when(i < 8)
        def _():
            ones1_v[pl.ds(i * 16, 16)] = one16

        @pl.when(i < 103)
        def _():
            zln_v[pl.ds(i * 16, 16)] = zero16
        return 0

    lax.fori_loop(0, 128, _init, 0)

    pltpu.sync_copy(zln_v.at[pl.ds(0, 1640)],
                    C1_sh.at[pl.ds(s * CSLICE, 1640)])
    pltpu.sync_copy(zln_v.at[pl.ds(0, 1640)],
                    C1_sh.at[pl.ds(s * CSLICE + 1640, 1640)])

    cbase = c * SCR

    def _pfc(j, _):
        for tt in range(8):
            kk = j * 8 + tt
            ctgt2_v[j, pl.ds(16 * tt, 16)] = SCR + ((kk % 64) * 16) + lane
        return 0

    lax.fori_loop(0, NBATCH + 1, _pfc, 0)
    cur = zero16i
    for k in range(NPIECE):
        ids16 = ids_v[pl.ds(16 * k, 16)]
        m = (ids16 >= cbase) & (ids16 < cbase + SCR)
        mi = jnp.where(m, 1, 0).astype(jnp.int32)
        pos = cur + lax.cumsum(mi) - mi
        plsc.store_scatter(ctgt2_v, [pos >> 7, pos & 127],
                           ids16 - cbase, mask=m)
        cur = cur + plsc.all_reduce_population_count(m)
    nC = cur[0]

    plsc.subcore_barrier()

    for b in range(NBATCH):
        @pl.when(b * 128 < nC)
        def _():
            pltpu.sync_copy(ones1_v, C1_sh.at[ctgt2_v.at[b]], add=True)

    wbp = [None, None]

    for r in range(NPASS):
        base = cbase + r * R

        def _pfp(j, _):
            for tt in range(8):
                kk = j * 8 + tt
                cidx2_v[j, pl.ds(16 * tt, 16)] = zero16i
                ctgt2_v[j, pl.ds(16 * tt, 16)] = R + ((kk % 14) * 16) + lane
            return 0

        lax.fori_loop(0, NBATCH + 1, _pfp, 0)

        cursor = zero16i
        for k in range(NPIECE):
            ids16 = ids_v[pl.ds(16 * k, 16)]
            m = (ids16 >= base) & (ids16 < base + R)
            mi = jnp.where(m, 1, 0).astype(jnp.int32)
            pos = cursor + lax.cumsum(mi) - mi
            plsc.store_scatter(cidx2_v, [pos >> 7, pos & 127],
                               row0 + 16 * k + lane, mask=m)
            plsc.store_scatter(ctgt2_v, [pos >> 7, pos & 127],
                               ids16 - base, mask=m)
            cursor = cursor + plsc.all_reduce_population_count(m)
        n = cursor[0]

        for b in range(NBATCH):
            @pl.when(b * 128 < n)
            def _():
                pltpu.sync_copy(zeros_v, S_sh.at[ctgt2_v.at[b]])
        plsc.subcore_barrier()

        for b in range(NBATCH):
            @pl.when(b * 128 < n)
            def _():
                pltpu.sync_copy(zb_hbm.at[cidx2_v.at[b]], cbuf_v)
                pltpu.sync_copy(cbuf_v, S_sh.at[ctgt2_v.at[b]], add=True)
        plsc.subcore_barrier()

        def _drain(j, _):
            chunk = s + 16 * j

            @pl.when((chunk < NCHUNK) & (base + chunk * CHUNK < N_BATCH))
            def _():
                st = chunk * CHUNK
                pltpu.sync_copy(S_sh.at[pl.ds(st, CHUNK)], sdr_v)
                pltpu.sync_copy(C1_sh.at[pl.ds(r * R + st, CHUNK)], cdr1_v)
                pltpu.sync_copy(cent_hbm.at[pl.ds(base + st, CHUNK)],
                                cc_v.at[0])

                def _grp(ii, _):
                    rb = ii * 16
                    cnt16 = cdr1_v[pl.ds(rb, 16)]
                    inv16 = 0.1 / jnp.maximum(cnt16, 1.0)
                    pf16 = jnp.where(cnt16 > 0.0, 1.0, 0.0)
                    for l in range(16):
                        p = pf16[l] > 0.5
                        iv = inv16[l]
                        for g in range(4):
                            sv = sdr_v[rb + l, pl.ds(16 * g, 16)]
                            cv = cc_v[0, rb + l, pl.ds(16 * g, 16)]
                            cc_v[0, rb + l, pl.ds(16 * g, 16)] = jnp.where(
                                p, 0.9 * cv + iv * sv, cv)
                    return 0

                lax.fori_loop(0, CHUNK // 16, _grp, 0)
                pltpu.sync_copy(cc_v.at[0], out_hbm.at[pl.ds(base + st, CHUNK)])
            return 0

        lax.fori_loop(0, (NCHUNK + 15) // 16, _drain, 0)

        if r != NPASS - 1:
            plsc.subcore_barrier()

    for q in range(2):
        if wbp[q] is not None:
            h, cond = wbp[q]

            @pl.when(cond)
            def _():
                h.wait()


def kernel(zb, batch_ids, centers):
    mesh = plsc.VectorSubcoreMesh(core_axis_name="c", subcore_axis_name="s")
    run = pl.kernel(
        _body,
        out_type=jax.ShapeDtypeStruct((N_BATCH, DIM), jnp.float32),
        mesh=mesh,
        compiler_params=pltpu.CompilerParams(
            use_tc_tiling_on_sc=False, needs_layout_passes=False),
        scratch_types=[
            pltpu.VMEM_SHARED((SROWS, DIM), jnp.float32),
            pltpu.VMEM_SHARED((CWORDS,), jnp.float32),
            pltpu.VMEM((RPT,), jnp.int32),
            pltpu.VMEM((NBATCH + 1, 128), jnp.int32),
            pltpu.VMEM((NBATCH + 1, 128), jnp.int32),
            pltpu.VMEM((128, DIM), jnp.float32),
            pltpu.VMEM((128, DIM), jnp.float32),
            pltpu.VMEM((128,), jnp.float32),
            pltpu.VMEM((1648,), jnp.float32),
            pltpu.VMEM((CHUNK, DIM), jnp.float32),
            pltpu.VMEM((CHUNK,), jnp.float32),
            pltpu.VMEM((2, CHUNK, DIM), jnp.float32),
            pltpu.SemaphoreType.DMA,
            pltpu.SemaphoreType.DMA,
            pltpu.SemaphoreType.DMA,
            pltpu.SemaphoreType.DMA,
            pltpu.SemaphoreType.DMA,
            pltpu.SemaphoreType.DMA,
            pltpu.SemaphoreType.DMA,
            pltpu.SemaphoreType.DMA,
            pltpu.SemaphoreType.DMA,
            pltpu.SemaphoreType.DMA,
        ],
    )
    return run(zb, batch_ids.astype(jnp.int32), centers)
